# Initial kernel scaffold; baseline (speedup 1.0000x reference)
import jax
import jax.numpy as jnp
from jax.experimental import pallas as pl


def kernel(input, Wg, W1, W2):
    raise NotImplementedError("write your pallas kernel here")


# speedup vs baseline: 2.5709x; 2.5709x over previous
import jax
import jax.numpy as jnp
from jax.experimental import pallas as pl

T = 2048
D = 1024
F = 2048
E = 8
CAP = 640
FBLK = 1024
NFB = F // FBLK
BISECT_ITERS = 40


def _routing_body(x_ref, wg_ref, w_ref):
    x_hi = x_ref[...].astype(jnp.bfloat16)
    g_hi = wg_ref[...].astype(jnp.bfloat16)
    logits = jax.lax.dot_general(
        g_hi, x_hi, (((0,), (1,)), ((), ())),
        preferred_element_type=jnp.float32)
    m = jnp.max(logits, axis=0, keepdims=True)
    ex = jnp.exp(logits - m)
    probs = ex / jnp.sum(ex, axis=0, keepdims=True)

    m1 = jnp.max(probs, axis=0, keepdims=True)
    masked = jnp.where(probs == m1, -1.0, probs)
    m2 = jnp.max(masked, axis=0, keepdims=True)
    denom = m1 + m2
    comb = jnp.where(probs >= m2, probs / denom, 0.0)

    lo0 = jnp.zeros((E, 1), jnp.float32)
    hi0 = jnp.max(comb, axis=1, keepdims=True) + 1.0

    def body(_, carry):
        lo, hi = carry
        mid = 0.5 * (lo + hi)
        cnt = jnp.sum((comb >= mid).astype(jnp.float32), axis=1, keepdims=True)
        pred = cnt >= CAP
        return jnp.where(pred, mid, lo), jnp.where(pred, hi, mid)

    lo, hi = jax.lax.fori_loop(0, BISECT_ITERS, body, (lo0, hi0))
    w_ref[...] = jnp.where(comb >= lo, comb, 0.0)


def _ffn_body(w_ref, x_ref, w1_ref, w2_ref, out_ref):
    e = pl.program_id(0)
    fb = pl.program_id(1)

    @pl.when((e == 0) & (fb == 0))
    def _():
        out_ref[...] = jnp.zeros_like(out_ref)

    oh = (jax.lax.broadcasted_iota(jnp.int32, (E, 1), 0) == e).astype(jnp.float32)
    wcol = jnp.dot(w_ref[...], oh, preferred_element_type=jnp.float32)

    xb = x_ref[...].astype(jnp.bfloat16)
    w1 = w1_ref[0].astype(jnp.bfloat16)
    h = jnp.dot(xb, w1, preferred_element_type=jnp.float32)
    h = jnp.maximum(h, 0.0) * wcol
    w2 = w2_ref[0].astype(jnp.bfloat16)
    out_ref[...] += jnp.dot(h.astype(jnp.bfloat16), w2,
                            preferred_element_type=jnp.float32)


def kernel(input, Wg, W1, W2):
    x = input
    w_et = pl.pallas_call(
        _routing_body,
        out_shape=jax.ShapeDtypeStruct((E, T), jnp.float32),
    )(x, Wg)
    w_te = w_et.T

    out = pl.pallas_call(
        _ffn_body,
        grid=(E, NFB),
        in_specs=[
            pl.BlockSpec((T, E), lambda e, fb: (0, 0)),
            pl.BlockSpec((T, D), lambda e, fb: (0, 0)),
            pl.BlockSpec((1, D, FBLK), lambda e, fb: (e, 0, fb)),
            pl.BlockSpec((1, FBLK, D), lambda e, fb: (e, fb, 0)),
        ],
        out_specs=pl.BlockSpec((T, D), lambda e, fb: (0, 0)),
        out_shape=jax.ShapeDtypeStruct((T, D), jnp.float32),
    )(w_te, x, W1, W2)
    return out
